# baseline (speedup 1.0000x reference)
import functools

import jax
import jax.numpy as jnp
from jax import lax
from jax.experimental import pallas as pl
from jax.experimental.pallas import tpu as pltpu
from jax.experimental.pallas import tpu_sc as plsc


def _embed_call(B, D, num_cores, num_subcores):
    NW = num_cores * num_subcores
    b_per_w = B // NW
    mesh = plsc.VectorSubcoreMesh(core_axis_name="c", subcore_axis_name="s")

    @functools.partial(
        pl.kernel,
        mesh=mesh,
        out_type=jax.ShapeDtypeStruct((B, D), jnp.float32),
        scratch_types=[
            pltpu.VMEM((b_per_w,), jnp.int32),
            pltpu.VMEM((b_per_w, D), jnp.float32),
            pltpu.SemaphoreType.DMA,
        ],
        compiler_params=pltpu.CompilerParams(use_tc_tiling_on_sc=False),
    )
    def emb(table_hbm, idx_hbm, out_hbm, idx_v, rows_v, sem):
        wid = lax.axis_index("s") * num_cores + lax.axis_index("c")
        base = wid * b_per_w
        pltpu.sync_copy(idx_hbm.at[pl.ds(base, b_per_w)], idx_v)
        pltpu.async_copy(table_hbm.at[idx_v], rows_v, sem).wait()
        pltpu.sync_copy(rows_v, out_hbm.at[pl.ds(base, b_per_w)])

    return emb


def kernel(x, table):
    (B,) = x.shape
    V, D = table.shape
    info = plsc.get_sparse_core_info()
    emb = _embed_call(B, D, info.num_cores, info.num_subcores)
    return emb(table, x.astype(jnp.int32))


# speedup vs baseline: 1.7282x; 1.7282x over previous
import functools

import jax
import jax.numpy as jnp
from jax import lax
from jax.experimental import pallas as pl
from jax.experimental.pallas import tpu as pltpu
from jax.experimental.pallas import tpu_sc as plsc


def _embed_call(B, D, num_cores, num_subcores):
    NW = num_cores * num_subcores
    b_per_w = B // NW
    mesh = plsc.VectorSubcoreMesh(core_axis_name="c", subcore_axis_name="s")

    @functools.partial(
        pl.kernel,
        mesh=mesh,
        out_type=jax.ShapeDtypeStruct((B, D), jnp.float32),
        scratch_types=[
            pltpu.VMEM_SHARED((num_subcores, b_per_w), jnp.int32),
            pltpu.SMEM((b_per_w,), jnp.int32),
            pltpu.VMEM((b_per_w, D), jnp.float32),
            pltpu.SemaphoreType.DMA,
        ],
    )
    def emb(table_hbm, idx_hbm, out_hbm, idx_sh, idx_s, rows_v, sem):
        cid = lax.axis_index("c")
        sid = lax.axis_index("s")
        wid = sid * num_cores + cid
        base = wid * b_per_w
        pltpu.sync_copy(idx_hbm.at[pl.ds(base, b_per_w)], idx_sh.at[sid])
        pltpu.sync_copy(idx_sh.at[sid], idx_s)

        def fire(i, carry):
            r = idx_s[i]
            pltpu.async_copy(
                table_hbm.at[pl.ds(r, 1), :], rows_v.at[pl.ds(i, 1), :], sem
            )
            return carry

        lax.fori_loop(0, b_per_w, fire, 0)
        pltpu.make_async_copy(table_hbm.at[pl.ds(0, b_per_w), :], rows_v, sem).wait()
        pltpu.sync_copy(rows_v, out_hbm.at[pl.ds(base, b_per_w)])

    return emb


def kernel(x, table):
    (B,) = x.shape
    V, D = table.shape
    info = plsc.get_sparse_core_info()
    emb = _embed_call(B, D, info.num_cores, info.num_subcores)
    return emb(table, x.astype(jnp.int32))


# speedup vs baseline: 1.7312x; 1.0017x over previous
import functools

import jax
import jax.numpy as jnp
from jax import lax
from jax.experimental import pallas as pl
from jax.experimental.pallas import tpu as pltpu
from jax.experimental.pallas import tpu_sc as plsc


def _embed_call(B, D, num_cores, num_subcores):
    NW = num_cores * num_subcores
    b_per_w = B // NW
    mesh = plsc.VectorSubcoreMesh(core_axis_name="c", subcore_axis_name="s")

    @functools.partial(
        pl.kernel,
        mesh=mesh,
        out_type=jax.ShapeDtypeStruct((B, D), jnp.float32),
        scratch_types=[
            pltpu.VMEM_SHARED((num_subcores, b_per_w), jnp.int32),
            pltpu.SMEM((b_per_w,), jnp.int32),
            pltpu.VMEM((b_per_w, D), jnp.float32),
            pltpu.SemaphoreType.DMA,
        ],
        compiler_params=pltpu.CompilerParams(skip_device_barrier=True),
    )
    def emb(table_hbm, idx_hbm, out_hbm, idx_sh, idx_s, rows_v, sem):
        cid = lax.axis_index("c")
        sid = lax.axis_index("s")
        wid = sid * num_cores + cid
        base = wid * b_per_w
        pltpu.sync_copy(idx_hbm.at[pl.ds(base, b_per_w)], idx_sh.at[sid])
        pltpu.sync_copy(idx_sh.at[sid], idx_s)

        def fire(i, carry):
            r = idx_s[i]
            pltpu.async_copy(
                table_hbm.at[pl.ds(r, 1), :], rows_v.at[pl.ds(i, 1), :], sem
            )
            return carry

        lax.fori_loop(0, b_per_w, fire, 0)
        pltpu.make_async_copy(table_hbm.at[pl.ds(0, b_per_w), :], rows_v, sem).wait()
        pltpu.sync_copy(rows_v, out_hbm.at[pl.ds(base, b_per_w)])

    return emb


def kernel(x, table):
    (B,) = x.shape
    V, D = table.shape
    info = plsc.get_sparse_core_info()
    emb = _embed_call(B, D, info.num_cores, info.num_subcores)
    return emb(table, x.astype(jnp.int32))


# speedup vs baseline: 3.0486x; 1.7610x over previous
import functools

import jax
import jax.numpy as jnp
from jax import lax
from jax.experimental import pallas as pl
from jax.experimental.pallas import tpu as pltpu
from jax.experimental.pallas import tpu_sc as plsc

_NBUF = 8
_QW = 128


def _embed_call(B, D, num_cores, num_subcores):
    NW = num_cores * num_subcores
    b_per_w = B // NW
    mesh = plsc.VectorSubcoreMesh(core_axis_name="c", subcore_axis_name="s")
    n_q = b_per_w // _QW
    n_outer = b_per_w // _NBUF

    @functools.partial(
        pl.kernel,
        mesh=mesh,
        out_type=jax.ShapeDtypeStruct((D, B), jnp.float32),
        scratch_types=[
            pltpu.VMEM_SHARED((num_subcores, b_per_w), jnp.int32),
            pltpu.SMEM((b_per_w,), jnp.int32),
            pltpu.VMEM((_NBUF, D, 128), jnp.float32),
            pltpu.VMEM((D, _QW), jnp.float32),
            pltpu.SemaphoreType.DMA((_NBUF,)),
        ],
        compiler_params=pltpu.CompilerParams(needs_layout_passes=False),
    )
    def emb(tab_t_hbm, idx_hbm, out_t_hbm, idx_sh, idx_s, blk_v, stage_v, sems):
        cid = lax.axis_index("c")
        sid = lax.axis_index("s")
        wid = sid * num_cores + cid
        base = wid * b_per_w
        pltpu.sync_copy(idx_hbm.at[pl.ds(base, b_per_w)], idx_sh.at[sid])
        pltpu.sync_copy(idx_sh.at[sid], idx_s)

        def fire(i, slot):
            c_off = pl.multiple_of(
                (idx_s[i] >> 7) * 128, 128
            )
            pltpu.async_copy(
                tab_t_hbm.at[:, pl.ds(c_off, 128)], blk_v.at[slot], sems.at[slot]
            )

        for b in range(_NBUF):
            fire(b, b)

        lanes = lax.iota(jnp.int32, 16)

        def outer(q, carry):
            for b in range(_NBUF):
                i = q * _NBUF + b
                pltpu.make_async_copy(
                    tab_t_hbm.at[:, pl.ds(0, 128)], blk_v.at[b], sems.at[b]
                ).wait()
                m = idx_s[i] & 127
                j = i % _QW
                m_v = jnp.full((16,), 0, jnp.int32) + m
                j_v = jnp.full((16,), 0, jnp.int32) + j
                for k in range(D // 16):
                    rows = lanes + (16 * k)
                    v = plsc.load_gather(blk_v.at[b], [rows, m_v])
                    plsc.store_scatter(stage_v, [rows, j_v], v)
                @pl.when(q < n_outer - 1)
                def _():
                    fire(i + _NBUF, b)

            @pl.when((q + 1) % (_QW // _NBUF) == 0)
            def _():
                q_off = pl.multiple_of(
                    base + ((q + 1) * _NBUF - _QW), 128
                )
                pltpu.sync_copy(stage_v, out_t_hbm.at[:, pl.ds(q_off, _QW)])

            return carry

        lax.fori_loop(0, n_outer, outer, 0)

    return emb


def kernel(x, table):
    (B,) = x.shape
    V, D = table.shape
    info = plsc.get_sparse_core_info()
    emb = _embed_call(B, D, info.num_cores, info.num_subcores)
    out_t = emb(table.T, x.astype(jnp.int32))
    return out_t.T
